# baseline (speedup 1.0000x reference)
import functools

import jax
import jax.numpy as jnp
from jax import lax
from jax.experimental import pallas as pl
from jax.experimental.pallas import tpu as pltpu
from jax.experimental.pallas import tpu_sc as plsc

VOCAB = 1000000
LSEQ = 200
D = 64
BATCH = 4096

NC = 2
NS = 16
NW = NC * NS

TOT = BATCH * LSEQ
RW = TOT // NW
S = LSEQ
NCH = RW // S
G = 100
NG = S // G
IDXROWS = TOT // G
NBUF = 2
NGRP = NCH // NBUF


def _sc_body(tok_hbm, idx_hbm, pos_hbm, out_hbm,
             idx0, idx1, in0, in1, out0, out1, pos_v,
             gsem0, gsem1, ssem0, ssem1):
    idx_v = (idx0, idx1)
    rin = (in0, in1)
    rout = (out0, out1)
    gsem = (gsem0, gsem1)
    ssem = (ssem0, ssem1)

    wid = lax.axis_index("s") * NC + lax.axis_index("c")
    base_row = wid * RW
    base_irow = wid * (RW // G)

    pltpu.sync_copy(pos_hbm, pos_v)

    def fetch(b, c):
        pltpu.sync_copy(idx_hbm.at[pl.ds(base_irow + c * NG, NG)], idx_v[b])
        for j in range(NG):
            pltpu.async_copy(tok_hbm.at[idx_v[b].at[j]],
                             rin[b].at[pl.ds(j * G, G)], gsem[b])

    for b in range(NBUF):
        fetch(b, b)

    @pl.loop(0, NGRP)
    def _grp(g):
        for b in range(NBUF):
            c = g * NBUF + b
            pltpu.make_async_copy(tok_hbm.at[pl.ds(0, S)], rin[b],
                                  gsem[b]).wait()
            @pl.when(g > 0)
            def _():
                pltpu.make_async_copy(
                    rout[b], out_hbm.at[pl.ds(0, S)], ssem[b]).wait()

            @pl.loop(0, S, unroll=8)
            def _row(r):
                for e in range(D // 16):
                    sl = pl.ds(e * 16, 16)
                    rout[b][r, sl] = rin[b][r, sl] + pos_v[r, sl]

            pltpu.async_copy(rout[b],
                             out_hbm.at[pl.ds(base_row + c * S, S)], ssem[b])

            @pl.when(g + 1 < NGRP)
            def _():
                fetch(b, c + NBUF)

    for b in range(NBUF):
        pltpu.make_async_copy(rout[b], out_hbm.at[pl.ds(0, S)],
                              ssem[b]).wait()


@jax.jit
def _sc_embed(xf2, token_table, pos_table):
    mesh = plsc.VectorSubcoreMesh(core_axis_name="c", subcore_axis_name="s")
    fn = pl.kernel(
        _sc_body,
        out_type=jax.ShapeDtypeStruct((TOT, D), jnp.float32),
        mesh=mesh,
        scratch_types=[
            pltpu.VMEM((NG, G), jnp.int32),
            pltpu.VMEM((NG, G), jnp.int32),
            pltpu.VMEM((S, D), jnp.float32),
            pltpu.VMEM((S, D), jnp.float32),
            pltpu.VMEM((S, D), jnp.float32),
            pltpu.VMEM((S, D), jnp.float32),
            pltpu.VMEM((S, D), jnp.float32),
            pltpu.SemaphoreType.DMA,
            pltpu.SemaphoreType.DMA,
            pltpu.SemaphoreType.DMA,
            pltpu.SemaphoreType.DMA,
        ],
        compiler_params=pltpu.CompilerParams(use_tc_tiling_on_sc=False),
    )
    return fn(token_table, xf2, pos_table)


def kernel(x, token_table, pos_table):
    xf2 = x.reshape(IDXROWS, G).astype(jnp.int32)
    out = _sc_embed(xf2, token_table, pos_table)
    return out.reshape(BATCH, LSEQ, D)
